# baseline (speedup 1.0000x reference)
import functools

import jax
import jax.numpy as jnp
from jax import lax
from jax.experimental import pallas as pl
from jax.experimental.pallas import tpu as pltpu
from jax.experimental.pallas import tpu_sc as plsc

NC = 2
NS = 16
NW = NC * NS
L = 16

F = 7
FA = F + 1


def _sc_segment_sums(xt, batch, n, n_seg):
    npw = n // NW
    npl = (npw + L - 1) // L
    w = ((npw + 16 + 7) // 8) * 8

    mesh = plsc.VectorSubcoreMesh(
        core_axis_name="c", subcore_axis_name="s",
        num_cores=NC, num_subcores=NS)

    @functools.partial(
        pl.kernel,
        out_type=jax.ShapeDtypeStruct((NW, FA, n_seg), jnp.float32),
        mesh=mesh,
        scratch_types=[
            pltpu.VMEM((F, w), jnp.float32),
            pltpu.VMEM((w,), jnp.int32),
            pltpu.VMEM((FA * n_seg,), jnp.float32),
        ],
        compiler_params=pltpu.CompilerParams(
            needs_layout_passes=False, use_tc_tiling_on_sc=False),
    )
    def seg_kernel(x_hbm, b_hbm, out_hbm, xv, bv, acc):
        wid = lax.axis_index("s") * NC + lax.axis_index("c")
        base = wid * npw
        b0 = (base // 8) * 8
        doff = base - b0
        for f in range(F):
            pltpu.sync_copy(x_hbm.at[f, pl.ds(b0, w)], xv.at[f])
        pltpu.sync_copy(b_hbm.at[pl.ds(b0, w)], bv)

        zeros = jnp.zeros((L,), jnp.float32)

        @plsc.parallel_loop(0, FA * n_seg // L, unroll=8)
        def _(i):
            acc[pl.ds(i * L, L)] = zeros

        lane = jnp.arange(L, dtype=jnp.int32)
        lane_node = lane * npl
        ones = jnp.ones((L,), jnp.float32)
        fsplat = [jnp.full((L,), f, jnp.int32) for f in range(F)]

        @plsc.parallel_loop(0, npl, unroll=7)
        def _(j):
            node = lane_node + j
            valid = node < npw
            nd = node + doff
            seg = plsc.load_gather(bv, [nd], mask=valid)
            plsc.addupdate_scatter(acc, [seg + (F * n_seg)], ones, mask=valid)
            for f in range(F):
                vals = plsc.load_gather(xv, [fsplat[f], nd], mask=valid)
                plsc.addupdate_scatter(acc, [seg + (f * n_seg)], vals,
                                       mask=valid)

        for f in range(FA):
            pltpu.sync_copy(acc.at[pl.ds(f * n_seg, n_seg)],
                            out_hbm.at[wid, f])

    return seg_kernel(xt, batch)


def _tc_finish(partials, u, w1u, w1m, b1, w2, b2):
    n_seg = u.shape[0]

    def body(p_ref, u_ref, w1u_ref, w1m_ref, b1_ref, w2_ref, b2_ref, o_ref):
        s = p_ref[0]
        for i in range(1, NW):
            s = s + p_ref[i]
        mean_t = s[:F, :] / jnp.maximum(s[F:FA, :], 1.0)
        h = jnp.dot(u_ref[...], w1u_ref[...], preferred_element_type=jnp.float32)
        h = h + lax.dot_general(mean_t, w1m_ref[...], (((0,), (0,)), ((), ())),
                                preferred_element_type=jnp.float32)
        h = h + b1_ref[...]
        h = jnp.where(h >= 0, h, 0.01 * h)
        o_ref[...] = (jnp.dot(h, w2_ref[...], preferred_element_type=jnp.float32)
                      + b2_ref[...])

    return pl.pallas_call(
        body,
        out_shape=jax.ShapeDtypeStruct((n_seg, w2.shape[1]), jnp.float32),
    )(partials, u, w1u, w1m, b1, w2, b2)


def kernel(x, edge_index, edge_attr, u, batch, W1, b1, W2, b2):
    n_seg = u.shape[0]
    gf = u.shape[1]

    n = x.shape[0]
    xt = jnp.pad(x.T, ((0, 0), (0, 16)))
    bpad = jnp.pad(batch.astype(jnp.int32), (0, 16))
    partials = _sc_segment_sums(xt, bpad, n, n_seg)

    w1u = W1[:gf]
    w1m = W1[gf:]
    return _tc_finish(partials, u, w1u, w1m,
                      b1.reshape(1, -1), W2, b2.reshape(1, -1))


# speedup vs baseline: 1.0342x; 1.0342x over previous
import functools

import jax
import jax.numpy as jnp
from jax import lax
from jax.experimental import pallas as pl
from jax.experimental.pallas import tpu as pltpu
from jax.experimental.pallas import tpu_sc as plsc

NC = 2
NS = 16
NW = NC * NS
L = 16

F = 7
FA = F + 1


def _sc_segment_sums(xt, batch, n, n_seg):
    npw = n // NW
    npl = (npw + L - 1) // L
    w = ((npw + 16 + 7) // 8) * 8

    mesh = plsc.VectorSubcoreMesh(
        core_axis_name="c", subcore_axis_name="s",
        num_cores=NC, num_subcores=NS)

    @functools.partial(
        pl.kernel,
        out_type=jax.ShapeDtypeStruct((NW, FA, n_seg), jnp.float32),
        mesh=mesh,
        scratch_types=[
            pltpu.VMEM((F, w), jnp.float32),
            pltpu.VMEM((w,), jnp.int32),
            pltpu.VMEM((FA * n_seg,), jnp.float32),
        ],
        compiler_params=pltpu.CompilerParams(
            needs_layout_passes=False, use_tc_tiling_on_sc=False),
    )
    def seg_kernel(x_hbm, b_hbm, out_hbm, xv, bv, acc):
        wid = lax.axis_index("s") * NC + lax.axis_index("c")
        base = wid * npw
        b0 = jnp.minimum((base // 8) * 8, n - w)
        doff = base - b0
        for f in range(F):
            pltpu.sync_copy(x_hbm.at[f, pl.ds(b0, w)], xv.at[f])
        pltpu.sync_copy(b_hbm.at[pl.ds(b0, w)], bv)

        zeros = jnp.zeros((L,), jnp.float32)

        @plsc.parallel_loop(0, FA * n_seg // L, unroll=8)
        def _(i):
            acc[pl.ds(i * L, L)] = zeros

        lane = jnp.arange(L, dtype=jnp.int32)
        lane_node = lane * npl
        ones = jnp.ones((L,), jnp.float32)
        fsplat = [jnp.full((L,), f, jnp.int32) for f in range(F)]

        def step(j):
            node = lane_node + j
            valid = node < npw
            nd = jnp.minimum(node, npw - 1) + doff
            seg = plsc.load_gather(bv, [nd], mask=valid)
            plsc.addupdate_scatter(acc, [seg + (F * n_seg)], ones, mask=valid)
            for f in range(F):
                vals = plsc.load_gather(xv, [fsplat[f], nd], mask=valid)
                plsc.addupdate_scatter(acc, [seg + (f * n_seg)], vals,
                                       mask=valid)

        half = npl // 2

        @plsc.parallel_loop(0, half, unroll=4)
        def _(j):
            step(j)
            step(j + half)

        if npl % 2:
            @plsc.parallel_loop(npl - 1, npl)
            def _(j):
                step(j)

        for f in range(FA):
            pltpu.sync_copy(acc.at[pl.ds(f * n_seg, n_seg)],
                            out_hbm.at[wid, f])

    return seg_kernel(xt, batch)


def _tc_finish(partials, u, w1u, w1m, b1, w2, b2):
    n_seg = u.shape[0]

    def body(p_ref, u_ref, w1u_ref, w1m_ref, b1_ref, w2_ref, b2_ref, o_ref):
        s = p_ref[0]
        for i in range(1, NW):
            s = s + p_ref[i]
        mean_t = s[:F, :] / jnp.maximum(s[F:FA, :], 1.0)
        h = jnp.dot(u_ref[...], w1u_ref[...], preferred_element_type=jnp.float32)
        h = h + lax.dot_general(mean_t, w1m_ref[...], (((0,), (0,)), ((), ())),
                                preferred_element_type=jnp.float32)
        h = h + b1_ref[...]
        h = jnp.where(h >= 0, h, 0.01 * h)
        o_ref[...] = (jnp.dot(h, w2_ref[...], preferred_element_type=jnp.float32)
                      + b2_ref[...])

    return pl.pallas_call(
        body,
        out_shape=jax.ShapeDtypeStruct((n_seg, w2.shape[1]), jnp.float32),
    )(partials, u, w1u, w1m, b1, w2, b2)


def kernel(x, edge_index, edge_attr, u, batch, W1, b1, W2, b2):
    n_seg = u.shape[0]
    gf = u.shape[1]

    n = x.shape[0]
    partials = _sc_segment_sums(x.T, batch.astype(jnp.int32), n, n_seg)

    w1u = W1[:gf]
    w1m = W1[gf:]
    return _tc_finish(partials, u, w1u, w1m,
                      b1.reshape(1, -1), W2, b2.reshape(1, -1))
